# baseline (speedup 1.0000x reference)
import functools

import jax
import jax.numpy as jnp
from jax.experimental import pallas as pl
from jax.experimental.pallas import tpu as pltpu


def _mha_body(x_ref, wqkv_ref, wpt_ref, bp_ref, o_ref, *, num_heads):
    T, C = x_ref.shape
    hs = C // num_heads

    x = x_ref[...].astype(jnp.bfloat16)

    qkv = jnp.dot(x, wqkv_ref[...], preferred_element_type=jnp.float32)
    qkv = qkv.astype(jnp.bfloat16)

    row = jax.lax.broadcasted_iota(jnp.int32, (T, T), 0)
    col = jax.lax.broadcasted_iota(jnp.int32, (T, T), 1)
    causal = col <= row
    neg_big = jnp.float32(-1e30)

    head_outs = []
    for h in range(num_heads):
        q = qkv[:, h * hs:(h + 1) * hs]
        k = qkv[:, C + h * hs:C + (h + 1) * hs]
        v = qkv[:, 2 * C + h * hs:2 * C + (h + 1) * hs]

        s = jax.lax.dot_general(q, k,
                                dimension_numbers=(((1,), (1,)), ((), ())),
                                preferred_element_type=jnp.float32)
        s = jnp.where(causal, s, neg_big)
        s = s - jnp.max(s, axis=-1, keepdims=True)
        p = jnp.exp(s)
        p = p * pl.reciprocal(jnp.sum(p, axis=-1, keepdims=True), approx=True)

        head_outs.append(jnp.dot(p.astype(jnp.bfloat16), v,
                                 preferred_element_type=jnp.float32))

    cat = jnp.concatenate(head_outs, axis=-1).astype(jnp.bfloat16)
    proj = jnp.dot(cat, wpt_ref[...], preferred_element_type=jnp.float32)
    o_ref[...] = (proj + bp_ref[...].astype(jnp.float32)).astype(o_ref.dtype)


@functools.partial(jax.jit, static_argnames=("num_heads",))
def _mha(x, wqkv_bf, wpt_bf, bp_f32, *, num_heads):
    B, T, C = x.shape

    body = functools.partial(_mha_body, num_heads=num_heads)
    out = pl.pallas_call(
        body,
        out_shape=jax.ShapeDtypeStruct((B * T, C), jnp.float32),
        grid=(B,),
        in_specs=[
            pl.BlockSpec((T, C), lambda i: (i, 0)),
            pl.BlockSpec((C, 3 * C), lambda i: (0, 0)),
            pl.BlockSpec((C, C), lambda i: (0, 0)),
            pl.BlockSpec((1, C), lambda i: (0, 0)),
        ],
        out_specs=pl.BlockSpec((T, C), lambda i: (i, 0)),
        compiler_params=pltpu.CompilerParams(
            dimension_semantics=("parallel",)),
        name="mha_blockdiag",
    )(x.reshape(B * T, C), wqkv_bf, wpt_bf, bp_f32)

    return out.reshape(B, T, C)


def kernel(x, wqkv_bf, wpt_bf, bp_f32):
    return _mha(x, wqkv_bf, wpt_bf, bp_f32, num_heads=12)


# speedup vs baseline: 1.6418x; 1.6418x over previous
import functools

import jax
import jax.numpy as jnp
from jax.experimental import pallas as pl
from jax.experimental.pallas import tpu as pltpu


def _mha_body(x_ref, wqkv_ref, wpt_ref, bp_ref, o_ref, *, num_heads, seq_len):
    R, C = x_ref.shape
    hs = C // num_heads
    T = seq_len

    x = x_ref[...].astype(jnp.bfloat16)

    qkv = jnp.dot(x, wqkv_ref[...], preferred_element_type=jnp.float32)
    qkv = qkv.astype(jnp.bfloat16)

    row = jax.lax.broadcasted_iota(jnp.int32, (R, R), 0)
    col = jax.lax.broadcasted_iota(jnp.int32, (R, R), 1)
    causal = (col <= row) & ((row // T) == (col // T))
    neg_big = jnp.float32(-1e30)

    head_outs = []
    for h in range(num_heads):
        q = qkv[:, h * hs:(h + 1) * hs]
        k = qkv[:, C + h * hs:C + (h + 1) * hs]
        v = qkv[:, 2 * C + h * hs:2 * C + (h + 1) * hs]

        s = jax.lax.dot_general(q, k,
                                dimension_numbers=(((1,), (1,)), ((), ())),
                                preferred_element_type=jnp.float32)
        s = jnp.where(causal, s, neg_big)
        s = s - jnp.max(s, axis=-1, keepdims=True)
        p = jnp.exp(s)
        p = p * pl.reciprocal(jnp.sum(p, axis=-1, keepdims=True), approx=True)

        head_outs.append(jnp.dot(p.astype(jnp.bfloat16), v,
                                 preferred_element_type=jnp.float32))

    cat = jnp.concatenate(head_outs, axis=-1).astype(jnp.bfloat16)
    proj = jnp.dot(cat, wpt_ref[...], preferred_element_type=jnp.float32)
    o_ref[...] = (proj + bp_ref[...].astype(jnp.float32)).astype(o_ref.dtype)


@functools.partial(jax.jit, static_argnames=("num_heads", "seqs_per_step"))
def _mha(x, wqkv_bf, wpt_bf, bp_f32, *, num_heads, seqs_per_step):
    B, T, C = x.shape
    R = seqs_per_step * T
    n_steps = B // seqs_per_step

    body = functools.partial(_mha_body, num_heads=num_heads, seq_len=T)
    out = pl.pallas_call(
        body,
        out_shape=jax.ShapeDtypeStruct((B * T, C), jnp.float32),
        grid=(n_steps,),
        in_specs=[
            pl.BlockSpec((R, C), lambda i: (i, 0)),
            pl.BlockSpec((C, 3 * C), lambda i: (0, 0)),
            pl.BlockSpec((C, C), lambda i: (0, 0)),
            pl.BlockSpec((1, C), lambda i: (0, 0)),
        ],
        out_specs=pl.BlockSpec((R, C), lambda i: (i, 0)),
        compiler_params=pltpu.CompilerParams(
            dimension_semantics=("parallel",)),
        name="mha_blockdiag",
    )(x.reshape(B * T, C), wqkv_bf, wpt_bf, bp_f32)

    return out.reshape(B, T, C)


def kernel(x, wqkv_bf, wpt_bf, bp_f32):
    return _mha(x, wqkv_bf, wpt_bf, bp_f32, num_heads=12, seqs_per_step=2)
